# baseline (device time: 218859 ns/iter reference)
import jax
import jax.numpy as jnp
from jax import lax
from jax.experimental import pallas as pl
from jax.experimental.pallas import tpu as pltpu

N_DEV = 4
M = 8192
D = 2048
CHUNK = M // N_DEV
W = D // 4
P = 4
TRS = (640, 640, 512, 256)
OFF = (0, 640, 1280, 1792)
TRMAX = max(TRS)
S = N_DEV - 1

_ORDER = [(0, 0), (1, 0), (0, 1), (1, 1), (0, 2), (1, 2),
          (2, 0), (3, 0), (2, 1), (3, 1), (2, 2), (3, 2)]


def kernel(partial, gamma):
    partial = partial.reshape(M, D)
    gamma2d = gamma.reshape(1, D)

    def body(partial_ref, gamma_ref, out_ref, comm_ref, local_ref,
             zsend, zrecv, xsend, xrecv, ysend, yrecv, local_sem):
        my_x = lax.axis_index("x")
        my_y = lax.axis_index("y")
        my_z = lax.axis_index("z")
        left = (my_z - 1) % N_DEV
        right = (my_z + 1) % N_DEV
        q = my_x + 2 * my_y

        barrier_sem = pltpu.get_barrier_semaphore()
        for dev in (
            (my_x, my_y, left),
            (my_x, my_y, right),
            (1 - my_x, my_y, my_z),
            (my_x, 1 - my_y, my_z),
        ):
            pl.semaphore_signal(
                barrier_sem, inc=1,
                device_id=dev, device_id_type=pl.DeviceIdType.MESH,
            )
        pl.semaphore_wait(barrier_sem, 4)

        def z_rdma(p, s):
            if s == 0:
                c = (my_z - 1) % N_DEV
                src = partial_ref.at[
                    pl.ds(c * CHUNK + OFF[p], TRS[p]), pl.ds(q * W, W)
                ]
            else:
                src = comm_ref.at[p, s - 1, :TRS[p]]
            return pltpu.make_async_remote_copy(
                src_ref=src,
                dst_ref=comm_ref.at[p, s, :TRS[p]],
                send_sem=zsend.at[p, s],
                recv_sem=zrecv.at[p, s],
                device_id=(my_x, my_y, right),
                device_id_type=pl.DeviceIdType.MESH,
            )

        def cp_make(p, s):
            c = (my_z - s - 2) % N_DEV
            return pltpu.make_async_copy(
                partial_ref.at[
                    pl.ds(c * CHUNK + OFF[p], TRS[p]), pl.ds(q * W, W)
                ],
                local_ref.at[:TRS[p]], local_sem,
            )

        def xchg_make(p):
            sl = out_ref.at[pl.ds(OFF[p], TRS[p]), pl.ds(q * W, W)]
            return pltpu.make_async_remote_copy(
                src_ref=sl, dst_ref=sl,
                send_sem=xsend.at[p], recv_sem=xrecv.at[p],
                device_id=(1 - my_x, my_y, my_z),
                device_id_type=pl.DeviceIdType.MESH,
            )

        def ychg_make(p):
            sl = out_ref.at[pl.ds(OFF[p], TRS[p]), pl.ds(my_y * (2 * W), 2 * W)]
            return pltpu.make_async_remote_copy(
                src_ref=sl, dst_ref=sl,
                send_sem=ysend.at[p], recv_sem=yrecv.at[p],
                device_id=(my_x, 1 - my_y, my_z),
                device_id_type=pl.DeviceIdType.MESH,
            )

        def norm(p):
            y = out_ref[OFF[p]:OFF[p] + TRS[p], :]
            msq = jnp.mean(y * y, axis=-1, keepdims=True)
            out_ref[OFF[p]:OFF[p] + TRS[p], :] = (
                y * lax.rsqrt(msq + 1e-6) * gamma_ref[:, :]
            )

        rdmas, cps, xchgs, ychgs = {}, {}, {}, {}
        rdmas[(0, 0)] = z_rdma(0, 0)
        rdmas[(0, 0)].start()
        cps[(0, 0)] = cp_make(0, 0)
        cps[(0, 0)].start()
        rdmas[(1, 0)] = z_rdma(1, 0)
        rdmas[(1, 0)].start()

        for k, (p, s) in enumerate(_ORDER):
            rdmas[(p, s)].wait()
            cps[(p, s)].wait()
            if s < S - 1:
                comm_ref[p, s, :TRS[p], :] += local_ref[:TRS[p], :]
                rdmas[(p, s + 1)] = z_rdma(p, s + 1)
                rdmas[(p, s + 1)].start()
            else:
                out_ref[OFF[p]:OFF[p] + TRS[p], pl.ds(q * W, W)] = (
                    comm_ref[p, s, :TRS[p], :] + local_ref[:TRS[p], :]
                )
                xchgs[p] = xchg_make(p)
                xchgs[p].start()
            if (p, s) == (0, 1):
                rdmas[(2, 0)] = z_rdma(2, 0)
                rdmas[(2, 0)].start()
            if (p, s) == (1, 1):
                rdmas[(3, 0)] = z_rdma(3, 0)
                rdmas[(3, 0)].start()
            if k + 1 < len(_ORDER):
                pn, sn = _ORDER[k + 1]
                cps[(pn, sn)] = cp_make(pn, sn)
                cps[(pn, sn)].start()
            if (p, s) == (2, 0):
                xchgs[0].wait()
                ychgs[0] = ychg_make(0)
                ychgs[0].start()
            if (p, s) == (3, 0):
                xchgs[1].wait()
                ychgs[1] = ychg_make(1)
                ychgs[1].start()
            if (p, s) == (2, 1):
                ychgs[0].wait()
                norm(0)
            if (p, s) == (3, 1):
                ychgs[1].wait()
                norm(1)

        xchgs[2].wait()
        ychgs[2] = ychg_make(2)
        ychgs[2].start()
        xchgs[3].wait()
        ychgs[3] = ychg_make(3)
        ychgs[3].start()
        ychgs[2].wait()
        norm(2)
        ychgs[3].wait()
        norm(3)

    return pl.pallas_call(
        body,
        out_shape=jax.ShapeDtypeStruct((CHUNK, D), jnp.float32),
        in_specs=[
            pl.BlockSpec(memory_space=pl.ANY),
            pl.BlockSpec(memory_space=pltpu.VMEM),
        ],
        out_specs=pl.BlockSpec(memory_space=pltpu.VMEM),
        scratch_shapes=[
            pltpu.VMEM((P, S, TRMAX, W), jnp.float32),
            pltpu.VMEM((TRMAX, W), jnp.float32),
            pltpu.SemaphoreType.DMA((P, S)),
            pltpu.SemaphoreType.DMA((P, S)),
            pltpu.SemaphoreType.DMA((P,)),
            pltpu.SemaphoreType.DMA((P,)),
            pltpu.SemaphoreType.DMA((P,)),
            pltpu.SemaphoreType.DMA((P,)),
            pltpu.SemaphoreType.DMA,
        ],
        compiler_params=pltpu.CompilerParams(collective_id=0),
    )(partial, gamma2d)


# device time: 218037 ns/iter; 1.0038x vs baseline; 1.0038x over previous
import jax
import jax.numpy as jnp
from jax import lax
from jax.experimental import pallas as pl
from jax.experimental.pallas import tpu as pltpu

N_DEV = 4
M = 8192
D = 2048
CHUNK = M // N_DEV
W = D // 4
P = 4
TRS = (640, 640, 512, 256)
OFF = (0, 640, 1280, 1792)
TRMAX = max(TRS)
S = N_DEV - 1

_ORDER = [(0, 0), (1, 0), (0, 1), (1, 1), (0, 2), (1, 2),
          (2, 0), (3, 0), (2, 1), (3, 1), (2, 2), (3, 2)]


def kernel(partial, gamma):
    partial = partial.reshape(M, D)
    gamma2d = gamma.reshape(1, D)

    def body(partial_ref, gamma_ref, out_ref, comm_ref, local_ref,
             zsend, zrecv, xsend, xrecv, ysend, yrecv, local_sem):
        my_x = lax.axis_index("x")
        my_y = lax.axis_index("y")
        my_z = lax.axis_index("z")
        left = (my_z - 1) % N_DEV
        right = (my_z + 1) % N_DEV
        q = my_x + 2 * my_y

        barrier_sem = pltpu.get_barrier_semaphore()
        for dev in (
            (my_x, my_y, left),
            (my_x, my_y, right),
            (1 - my_x, my_y, my_z),
            (my_x, 1 - my_y, my_z),
        ):
            pl.semaphore_signal(
                barrier_sem, inc=1,
                device_id=dev, device_id_type=pl.DeviceIdType.MESH,
            )
        pl.semaphore_wait(barrier_sem, 4)

        def z_rdma(p, s):
            if s == 0:
                c = (my_z - 1) % N_DEV
                src = partial_ref.at[
                    pl.ds(c * CHUNK + OFF[p], TRS[p]), pl.ds(q * W, W)
                ]
            else:
                src = comm_ref.at[p, s - 1, :TRS[p]]
            return pltpu.make_async_remote_copy(
                src_ref=src,
                dst_ref=comm_ref.at[p, s, :TRS[p]],
                send_sem=zsend.at[p, s],
                recv_sem=zrecv.at[p, s],
                device_id=(my_x, my_y, right),
                device_id_type=pl.DeviceIdType.MESH,
            )

        def cp_make(p, s):
            c = (my_z - s - 2) % N_DEV
            return pltpu.make_async_copy(
                partial_ref.at[
                    pl.ds(c * CHUNK + OFF[p], TRS[p]), pl.ds(q * W, W)
                ],
                local_ref.at[:TRS[p]], local_sem,
            )

        def xchg_make(p):
            sl = out_ref.at[pl.ds(OFF[p], TRS[p]), pl.ds(q * W, W)]
            return pltpu.make_async_remote_copy(
                src_ref=sl, dst_ref=sl,
                send_sem=xsend.at[p], recv_sem=xrecv.at[p],
                device_id=(1 - my_x, my_y, my_z),
                device_id_type=pl.DeviceIdType.MESH,
            )

        def ychg_make(p):
            sl = out_ref.at[pl.ds(OFF[p], TRS[p]), pl.ds(my_y * (2 * W), 2 * W)]
            return pltpu.make_async_remote_copy(
                src_ref=sl, dst_ref=sl,
                send_sem=ysend.at[p], recv_sem=yrecv.at[p],
                device_id=(my_x, 1 - my_y, my_z),
                device_id_type=pl.DeviceIdType.MESH,
            )

        def norm(p):
            y = out_ref[OFF[p]:OFF[p] + TRS[p], :]
            msq = jnp.mean(y * y, axis=-1, keepdims=True)
            out_ref[OFF[p]:OFF[p] + TRS[p], :] = (
                y * lax.rsqrt(msq + 1e-6) * gamma_ref[:, :]
            )

        rdmas, cps, xchgs, ychgs = {}, {}, {}, {}
        rdmas[(0, 0)] = z_rdma(0, 0)
        rdmas[(0, 0)].start()
        cps[(0, 0)] = cp_make(0, 0)
        cps[(0, 0)].start()
        rdmas[(1, 0)] = z_rdma(1, 0)
        rdmas[(1, 0)].start()

        for k, (p, s) in enumerate(_ORDER):
            rdmas[(p, s)].wait()
            cps[(p, s)].wait()
            if s < S - 1:
                comm_ref[p, s, :TRS[p], :] += local_ref[:TRS[p], :]
                rdmas[(p, s + 1)] = z_rdma(p, s + 1)
                rdmas[(p, s + 1)].start()
            else:
                out_ref[OFF[p]:OFF[p] + TRS[p], pl.ds(q * W, W)] = (
                    comm_ref[p, s, :TRS[p], :] + local_ref[:TRS[p], :]
                )
                xchgs[p] = xchg_make(p)
                xchgs[p].start()
            if (p, s) == (0, 1):
                rdmas[(2, 0)] = z_rdma(2, 0)
                rdmas[(2, 0)].start()
            if (p, s) == (1, 1):
                rdmas[(3, 0)] = z_rdma(3, 0)
                rdmas[(3, 0)].start()
            if k + 1 < len(_ORDER):
                pn, sn = _ORDER[k + 1]
                cps[(pn, sn)] = cp_make(pn, sn)
                cps[(pn, sn)].start()
            if (p, s) == (2, 0):
                xchgs[0].wait()
                ychgs[0] = ychg_make(0)
                ychgs[0].start()
            if (p, s) == (3, 0):
                xchgs[1].wait()
                ychgs[1] = ychg_make(1)
                ychgs[1].start()
            if (p, s) == (2, 1):
                ychgs[0].wait()
            if (p, s) == (3, 1):
                ychgs[1].wait()

        xchgs[2].wait()
        ychgs[2] = ychg_make(2)
        ychgs[2].start()
        xchgs[3].wait()
        ychgs[3] = ychg_make(3)
        ychgs[3].start()
        norm(0)
        norm(1)
        ychgs[2].wait()
        norm(2)
        ychgs[3].wait()
        norm(3)

    return pl.pallas_call(
        body,
        out_shape=jax.ShapeDtypeStruct((CHUNK, D), jnp.float32),
        in_specs=[
            pl.BlockSpec(memory_space=pl.ANY),
            pl.BlockSpec(memory_space=pltpu.VMEM),
        ],
        out_specs=pl.BlockSpec(memory_space=pltpu.VMEM),
        scratch_shapes=[
            pltpu.VMEM((P, S, TRMAX, W), jnp.float32),
            pltpu.VMEM((TRMAX, W), jnp.float32),
            pltpu.SemaphoreType.DMA((P, S)),
            pltpu.SemaphoreType.DMA((P, S)),
            pltpu.SemaphoreType.DMA((P,)),
            pltpu.SemaphoreType.DMA((P,)),
            pltpu.SemaphoreType.DMA((P,)),
            pltpu.SemaphoreType.DMA((P,)),
            pltpu.SemaphoreType.DMA,
        ],
        compiler_params=pltpu.CompilerParams(collective_id=0),
    )(partial, gamma2d)


# device time: 201291 ns/iter; 1.0873x vs baseline; 1.0832x over previous
import jax
import jax.numpy as jnp
from jax import lax
from jax.experimental import pallas as pl
from jax.experimental.pallas import tpu as pltpu

N_DEV = 4
M = 8192
D = 2048
CHUNK = M // N_DEV
W = D // 4
P = 4
TRS = (512, 512, 512, 512)
OFF = (0, 512, 1024, 1536)
TRMAX = max(TRS)
S = N_DEV - 1

_ORDER = [(0, 0), (1, 0), (0, 1), (1, 1), (0, 2), (1, 2),
          (2, 0), (3, 0), (2, 1), (3, 1), (2, 2), (3, 2)]


def kernel(partial, gamma):
    partial = partial.reshape(M, D)
    gamma2d = gamma.reshape(1, D)

    def body(partial_ref, gamma_ref, out_ref, comm_ref, local_ref,
             zsend, zrecv, xsend, xrecv, ysend, yrecv, local_sem):
        my_x = lax.axis_index("x")
        my_y = lax.axis_index("y")
        my_z = lax.axis_index("z")
        left = (my_z - 1) % N_DEV
        right = (my_z + 1) % N_DEV
        q = my_x + 2 * my_y

        barrier_sem = pltpu.get_barrier_semaphore()
        for dev in (
            (my_x, my_y, left),
            (my_x, my_y, right),
            (1 - my_x, my_y, my_z),
            (my_x, 1 - my_y, my_z),
        ):
            pl.semaphore_signal(
                barrier_sem, inc=1,
                device_id=dev, device_id_type=pl.DeviceIdType.MESH,
            )
        pl.semaphore_wait(barrier_sem, 4)

        def z_rdma(p, s):
            if s == 0:
                c = (my_z - 1) % N_DEV
                src = partial_ref.at[
                    pl.ds(c * CHUNK + OFF[p], TRS[p]), pl.ds(q * W, W)
                ]
            else:
                src = comm_ref.at[p, s - 1, :TRS[p]]
            return pltpu.make_async_remote_copy(
                src_ref=src,
                dst_ref=comm_ref.at[p, s, :TRS[p]],
                send_sem=zsend.at[p, s],
                recv_sem=zrecv.at[p, s],
                device_id=(my_x, my_y, right),
                device_id_type=pl.DeviceIdType.MESH,
            )

        def cp_make(p, s):
            c = (my_z - s - 2) % N_DEV
            return pltpu.make_async_copy(
                partial_ref.at[
                    pl.ds(c * CHUNK + OFF[p], TRS[p]), pl.ds(q * W, W)
                ],
                local_ref.at[:TRS[p]], local_sem,
            )

        def xchg_make(p):
            sl = out_ref.at[pl.ds(OFF[p], TRS[p]), pl.ds(q * W, W)]
            return pltpu.make_async_remote_copy(
                src_ref=sl, dst_ref=sl,
                send_sem=xsend.at[p], recv_sem=xrecv.at[p],
                device_id=(1 - my_x, my_y, my_z),
                device_id_type=pl.DeviceIdType.MESH,
            )

        def ychg_make(p):
            sl = out_ref.at[pl.ds(OFF[p], TRS[p]), pl.ds(my_y * (2 * W), 2 * W)]
            return pltpu.make_async_remote_copy(
                src_ref=sl, dst_ref=sl,
                send_sem=ysend.at[p], recv_sem=yrecv.at[p],
                device_id=(my_x, 1 - my_y, my_z),
                device_id_type=pl.DeviceIdType.MESH,
            )

        def norm(p):
            y = out_ref[OFF[p]:OFF[p] + TRS[p], :]
            msq = jnp.mean(y * y, axis=-1, keepdims=True)
            out_ref[OFF[p]:OFF[p] + TRS[p], :] = (
                y * lax.rsqrt(msq + 1e-6) * gamma_ref[:, :]
            )

        rdmas, cps, xchgs, ychgs = {}, {}, {}, {}
        rdmas[(0, 0)] = z_rdma(0, 0)
        rdmas[(0, 0)].start()
        cps[(0, 0)] = cp_make(0, 0)
        cps[(0, 0)].start()
        rdmas[(1, 0)] = z_rdma(1, 0)
        rdmas[(1, 0)].start()

        for k, (p, s) in enumerate(_ORDER):
            rdmas[(p, s)].wait()
            cps[(p, s)].wait()
            if s < S - 1:
                comm_ref[p, s, :TRS[p], :] += local_ref[:TRS[p], :]
                rdmas[(p, s + 1)] = z_rdma(p, s + 1)
                rdmas[(p, s + 1)].start()
            else:
                out_ref[OFF[p]:OFF[p] + TRS[p], pl.ds(q * W, W)] = (
                    comm_ref[p, s, :TRS[p], :] + local_ref[:TRS[p], :]
                )
                xchgs[p] = xchg_make(p)
                xchgs[p].start()
            if (p, s) == (0, 1):
                rdmas[(2, 0)] = z_rdma(2, 0)
                rdmas[(2, 0)].start()
            if (p, s) == (1, 1):
                rdmas[(3, 0)] = z_rdma(3, 0)
                rdmas[(3, 0)].start()
            if k + 1 < len(_ORDER):
                pn, sn = _ORDER[k + 1]
                cps[(pn, sn)] = cp_make(pn, sn)
                cps[(pn, sn)].start()
            if (p, s) == (2, 0):
                xchgs[0].wait()
                ychgs[0] = ychg_make(0)
                ychgs[0].start()
            if (p, s) == (3, 0):
                xchgs[1].wait()
                ychgs[1] = ychg_make(1)
                ychgs[1].start()
            if (p, s) == (2, 1):
                ychgs[0].wait()
            if (p, s) == (3, 1):
                ychgs[1].wait()

        xchgs[2].wait()
        ychgs[2] = ychg_make(2)
        ychgs[2].start()
        xchgs[3].wait()
        ychgs[3] = ychg_make(3)
        ychgs[3].start()
        norm(0)
        norm(1)
        ychgs[2].wait()
        norm(2)
        ychgs[3].wait()
        norm(3)

    return pl.pallas_call(
        body,
        out_shape=jax.ShapeDtypeStruct((CHUNK, D), jnp.float32),
        in_specs=[
            pl.BlockSpec(memory_space=pl.ANY),
            pl.BlockSpec(memory_space=pltpu.VMEM),
        ],
        out_specs=pl.BlockSpec(memory_space=pltpu.VMEM),
        scratch_shapes=[
            pltpu.VMEM((P, S, TRMAX, W), jnp.float32),
            pltpu.VMEM((TRMAX, W), jnp.float32),
            pltpu.SemaphoreType.DMA((P, S)),
            pltpu.SemaphoreType.DMA((P, S)),
            pltpu.SemaphoreType.DMA((P,)),
            pltpu.SemaphoreType.DMA((P,)),
            pltpu.SemaphoreType.DMA((P,)),
            pltpu.SemaphoreType.DMA((P,)),
            pltpu.SemaphoreType.DMA,
        ],
        compiler_params=pltpu.CompilerParams(collective_id=0),
    )(partial, gamma2d)


# device time: 196895 ns/iter; 1.1116x vs baseline; 1.0223x over previous
import jax
import jax.numpy as jnp
from jax import lax
from jax.experimental import pallas as pl
from jax.experimental.pallas import tpu as pltpu

N_DEV = 4
M = 8192
D = 2048
CHUNK = M // N_DEV
W = D // 4
P = 4
TRS = (512, 512, 512, 512)
OFF = (0, 512, 1024, 1536)
TRMAX = max(TRS)
S = N_DEV - 1

_ORDER = [(0, 0), (1, 0), (0, 1), (1, 1), (0, 2), (1, 2),
          (2, 0), (3, 0), (2, 1), (3, 1), (2, 2), (3, 2)]


def kernel(partial, gamma):
    partial = partial.reshape(M, D)
    gamma2d = gamma.reshape(1, D)

    def body(partial_ref, gamma_ref, out_ref, comm_ref, local_ref,
             zsend, zrecv, xsend, xrecv, ysend, yrecv,
             ya_send, ya_recv, xb1_send, xb1_recv, xb2_send, xb2_recv,
             local_sem):
        my_x = lax.axis_index("x")
        my_y = lax.axis_index("y")
        my_z = lax.axis_index("z")
        left = (my_z - 1) % N_DEV
        right = (my_z + 1) % N_DEV
        q = my_x + 2 * my_y

        barrier_sem = pltpu.get_barrier_semaphore()
        for dev in (
            (my_x, my_y, left),
            (my_x, my_y, right),
            (1 - my_x, my_y, my_z),
            (my_x, 1 - my_y, my_z),
        ):
            pl.semaphore_signal(
                barrier_sem, inc=1,
                device_id=dev, device_id_type=pl.DeviceIdType.MESH,
            )
        pl.semaphore_wait(barrier_sem, 4)

        def z_rdma(p, s):
            if s == 0:
                c = (my_z - 1) % N_DEV
                src = partial_ref.at[
                    pl.ds(c * CHUNK + OFF[p], TRS[p]), pl.ds(q * W, W)
                ]
            else:
                src = comm_ref.at[p, s - 1, :TRS[p]]
            return pltpu.make_async_remote_copy(
                src_ref=src,
                dst_ref=comm_ref.at[p, s, :TRS[p]],
                send_sem=zsend.at[p, s],
                recv_sem=zrecv.at[p, s],
                device_id=(my_x, my_y, right),
                device_id_type=pl.DeviceIdType.MESH,
            )

        def cp_make(p, s):
            c = (my_z - s - 2) % N_DEV
            return pltpu.make_async_copy(
                partial_ref.at[
                    pl.ds(c * CHUNK + OFF[p], TRS[p]), pl.ds(q * W, W)
                ],
                local_ref.at[:TRS[p]], local_sem,
            )

        def xchg_make(p):
            sl = out_ref.at[pl.ds(OFF[p], TRS[p]), pl.ds(q * W, W)]
            return pltpu.make_async_remote_copy(
                src_ref=sl, dst_ref=sl,
                send_sem=xsend.at[p], recv_sem=xrecv.at[p],
                device_id=(1 - my_x, my_y, my_z),
                device_id_type=pl.DeviceIdType.MESH,
            )

        def ychg_make(p):
            sl = out_ref.at[pl.ds(OFF[p], TRS[p]), pl.ds(my_y * (2 * W), 2 * W)]
            return pltpu.make_async_remote_copy(
                src_ref=sl, dst_ref=sl,
                send_sem=ysend.at[p], recv_sem=yrecv.at[p],
                device_id=(my_x, 1 - my_y, my_z),
                device_id_type=pl.DeviceIdType.MESH,
            )

        HR = TRS[3] // 2

        def xa3_make():
            sl = out_ref.at[pl.ds(OFF[3], HR), pl.ds(q * W, W)]
            return pltpu.make_async_remote_copy(
                src_ref=sl, dst_ref=sl,
                send_sem=xsend.at[3], recv_sem=xrecv.at[3],
                device_id=(1 - my_x, my_y, my_z),
                device_id_type=pl.DeviceIdType.MESH,
            )

        def yb3_make():
            sl = out_ref.at[pl.ds(OFF[3] + HR, HR), pl.ds(q * W, W)]
            return pltpu.make_async_remote_copy(
                src_ref=sl, dst_ref=sl,
                send_sem=ysend.at[3], recv_sem=yrecv.at[3],
                device_id=(my_x, 1 - my_y, my_z),
                device_id_type=pl.DeviceIdType.MESH,
            )

        def ya3_make():
            sl = out_ref.at[pl.ds(OFF[3], HR), pl.ds(my_y * (2 * W), 2 * W)]
            return pltpu.make_async_remote_copy(
                src_ref=sl, dst_ref=sl,
                send_sem=ya_send, recv_sem=ya_recv,
                device_id=(my_x, 1 - my_y, my_z),
                device_id_type=pl.DeviceIdType.MESH,
            )

        def xb3_make(block, send_sem, recv_sem):
            sl = out_ref.at[pl.ds(OFF[3] + HR, HR), pl.ds(block * W, W)]
            return pltpu.make_async_remote_copy(
                src_ref=sl, dst_ref=sl,
                send_sem=send_sem, recv_sem=recv_sem,
                device_id=(1 - my_x, my_y, my_z),
                device_id_type=pl.DeviceIdType.MESH,
            )

        def norm(p):
            y = out_ref[OFF[p]:OFF[p] + TRS[p], :]
            msq = jnp.mean(y * y, axis=-1, keepdims=True)
            out_ref[OFF[p]:OFF[p] + TRS[p], :] = (
                y * lax.rsqrt(msq + 1e-6) * gamma_ref[:, :]
            )

        rdmas, cps, xchgs, ychgs = {}, {}, {}, {}
        rdmas[(0, 0)] = z_rdma(0, 0)
        rdmas[(0, 0)].start()
        cps[(0, 0)] = cp_make(0, 0)
        cps[(0, 0)].start()
        rdmas[(1, 0)] = z_rdma(1, 0)
        rdmas[(1, 0)].start()

        for k, (p, s) in enumerate(_ORDER):
            if (p, s) == (3, 2):
                xchgs[2].wait()
                ychgs[2] = ychg_make(2)
                ychgs[2].start()
            rdmas[(p, s)].wait()
            cps[(p, s)].wait()
            if s < S - 1:
                comm_ref[p, s, :TRS[p], :] += local_ref[:TRS[p], :]
                rdmas[(p, s + 1)] = z_rdma(p, s + 1)
                rdmas[(p, s + 1)].start()
            else:
                out_ref[OFF[p]:OFF[p] + TRS[p], pl.ds(q * W, W)] = (
                    comm_ref[p, s, :TRS[p], :] + local_ref[:TRS[p], :]
                )
                if p < 3:
                    xchgs[p] = xchg_make(p)
                    xchgs[p].start()
                else:
                    xa3 = xa3_make()
                    xa3.start()
                    yb3 = yb3_make()
                    yb3.start()
            if (p, s) == (0, 1):
                rdmas[(2, 0)] = z_rdma(2, 0)
                rdmas[(2, 0)].start()
            if (p, s) == (1, 1):
                rdmas[(3, 0)] = z_rdma(3, 0)
                rdmas[(3, 0)].start()
            if k + 1 < len(_ORDER):
                pn, sn = _ORDER[k + 1]
                cps[(pn, sn)] = cp_make(pn, sn)
                cps[(pn, sn)].start()
            if (p, s) == (2, 0):
                xchgs[0].wait()
                ychgs[0] = ychg_make(0)
                ychgs[0].start()
            if (p, s) == (3, 0):
                xchgs[1].wait()
                ychgs[1] = ychg_make(1)
                ychgs[1].start()
            if (p, s) == (2, 1):
                ychgs[0].wait()
            if (p, s) == (3, 1):
                ychgs[1].wait()

        xa3.wait()
        ya3 = ya3_make()
        ya3.start()
        yb3.wait()
        xb1 = xb3_make(q, xb1_send, xb1_recv)
        xb1.start()
        xb2 = xb3_make(q ^ 2, xb2_send, xb2_recv)
        xb2.start()
        norm(0)
        norm(1)
        ychgs[2].wait()
        norm(2)
        ya3.wait()
        xb1.wait()
        xb2.wait()
        norm(3)

    return pl.pallas_call(
        body,
        out_shape=jax.ShapeDtypeStruct((CHUNK, D), jnp.float32),
        in_specs=[
            pl.BlockSpec(memory_space=pl.ANY),
            pl.BlockSpec(memory_space=pltpu.VMEM),
        ],
        out_specs=pl.BlockSpec(memory_space=pltpu.VMEM),
        scratch_shapes=[
            pltpu.VMEM((P, S, TRMAX, W), jnp.float32),
            pltpu.VMEM((TRMAX, W), jnp.float32),
            pltpu.SemaphoreType.DMA((P, S)),
            pltpu.SemaphoreType.DMA((P, S)),
            pltpu.SemaphoreType.DMA((P,)),
            pltpu.SemaphoreType.DMA((P,)),
            pltpu.SemaphoreType.DMA((P,)),
            pltpu.SemaphoreType.DMA((P,)),
            pltpu.SemaphoreType.DMA,
            pltpu.SemaphoreType.DMA,
            pltpu.SemaphoreType.DMA,
            pltpu.SemaphoreType.DMA,
            pltpu.SemaphoreType.DMA,
            pltpu.SemaphoreType.DMA,
            pltpu.SemaphoreType.DMA,
        ],
        compiler_params=pltpu.CompilerParams(collective_id=0),
    )(partial, gamma2d)
